# baseline (speedup 1.0000x reference)
import functools

import jax
import jax.numpy as jnp
from jax import lax
from jax.experimental import pallas as pl
from jax.experimental.pallas import tpu as pltpu
from jax.experimental.pallas import tpu_sc as plsc

_DELTA = 0.5
_LOSS_WEIGHT = 0.5
_TOPK = 64
_SOFT = 6.0
_NEG_INF = float("-inf")
_K6 = _SOFT * 1.4426950408889634
_LN2 = 0.6931471805599453


def _rev16(v):
    return jnp.flip(v, axis=0)


def _merge16(a, b):
    rb = _rev16(b)
    return jnp.sort(jnp.minimum(a, rb)), jnp.sort(jnp.maximum(a, rb))


def _clean32(x0, x1):
    return jnp.sort(jnp.minimum(x0, x1)), jnp.sort(jnp.maximum(x0, x1))


def _sort64(v0, v1, v2, v3):
    lo0, hi0 = _merge16(jnp.sort(v0), jnp.sort(v1))
    lo1, hi1 = _merge16(jnp.sort(v2), jnp.sort(v3))
    rb0, rb1 = _rev16(hi1), _rev16(lo1)
    loa, lob = jnp.minimum(lo0, rb0), jnp.minimum(hi0, rb1)
    hia, hib = jnp.maximum(lo0, rb0), jnp.maximum(hi0, rb1)
    s0, s1 = _clean32(loa, lob)
    s2, s3 = _clean32(hia, hib)
    return s0, s1, s2, s3


def _merge64_top(a, b):
    z = [
        jnp.maximum(a[0], _rev16(b[3])),
        jnp.maximum(a[1], _rev16(b[2])),
        jnp.maximum(a[2], _rev16(b[1])),
        jnp.maximum(a[3], _rev16(b[0])),
    ]
    t0, t2 = jnp.minimum(z[0], z[2]), jnp.maximum(z[0], z[2])
    t1, t3 = jnp.minimum(z[1], z[3]), jnp.maximum(z[1], z[3])
    u0, u1 = jnp.minimum(t0, t1), jnp.maximum(t0, t1)
    u2, u3 = jnp.minimum(t2, t3), jnp.maximum(t2, t3)
    return jnp.sort(u0), jnp.sort(u1), jnp.sort(u2), jnp.sort(u3)


def _sc_topk_body(pred_hbm, tgt_hbm, out_hbm, prow, trow, bufa, bufb, orow):
    nc_sc = 2
    wid = lax.axis_index("s") * nc_sc + lax.axis_index("c")
    neg_inf = jnp.full((16,), _NEG_INF, jnp.float32)

    def load4(ref, base):
        return [ref[pl.ds(base + 16 * j, 16)] for j in range(4)]

    def store4(ref, base, vs):
        for j in range(4):
            ref[pl.ds(base + 16 * j, 16)] = vs[j]

    for half in range(2):
        r = wid + 32 * half
        b = r // 16
        pltpu.sync_copy(pred_hbm.at[r], prow)
        pltpu.sync_copy(tgt_hbm.at[b], trow)

        def build(i, carry):
            base = i * 64
            vs = []
            for j in range(4):
                v = prow[pl.ds(base + 16 * j, 16)]
                t = trow[pl.ds(base + 16 * j, 16)]
                vs.append(jnp.where(t == 16, v, neg_inf))
            store4(bufa, base, list(_sort64(*vs)))
            return carry

        lax.fori_loop(0, 32, build, 0)

        def make_merge(src, dst):
            def mrg(i, carry):
                a = load4(src, (2 * i) * 64)
                b4 = load4(src, (2 * i + 1) * 64)
                store4(dst, i * 64, list(_merge64_top(a, b4)))
                return carry

            return mrg

        lax.fori_loop(0, 16, make_merge(bufa, bufb), 0)
        lax.fori_loop(0, 8, make_merge(bufb, bufa), 0)
        lax.fori_loop(0, 4, make_merge(bufa, bufb), 0)
        lax.fori_loop(0, 2, make_merge(bufb, bufa), 0)
        lax.fori_loop(0, 1, make_merge(bufa, bufb), 0)

        for j in range(4):
            v = bufb[pl.ds(16 * j, 16)]
            orow[pl.ds(16 * j, 16)] = jnp.where(
                v == _NEG_INF, jnp.nan, _K6 * (v + _DELTA)
            )
        pltpu.sync_copy(orow, out_hbm.at[r])


@functools.cache
def _sc_topk():
    return pl.kernel(
        _sc_topk_body,
        out_type=jax.ShapeDtypeStruct((64, 64), jnp.float32),
        mesh=plsc.VectorSubcoreMesh(core_axis_name="c", subcore_axis_name="s"),
        scratch_types=[
            pltpu.VMEM((2048,), jnp.float32),
            pltpu.VMEM((2048,), jnp.int32),
            pltpu.VMEM((2048,), jnp.float32),
            pltpu.VMEM((1024,), jnp.float32),
            pltpu.VMEM((64,), jnp.float32),
        ],
        compiler_params=pltpu.CompilerParams(needs_layout_passes=False),
    )


def _pair_body(tp_ref, tgt_ref, a2_ref, s_ref, np_ref):
    step = pl.program_id(0)
    r, c = tp_ref.shape
    nb, k = a2_ref.shape
    bk = (nb // c) * k
    tp6 = tp_ref[...] * _K6
    acc = jnp.zeros((r, bk), jnp.float32)
    for cc in range(c):
        a2 = jnp.concatenate(
            [a2_ref[b2 * c + cc : b2 * c + cc + 1, :] for b2 in range(nb // c)],
            axis=1,
        )
        x2 = a2 - tp6[:, cc : cc + 1]
        acc = acc + jnp.log(1.0 + jnp.exp2(x2))
    rows = jnp.sum(acc, axis=1)
    m = (tgt_ref[...][:, 0] != c).astype(jnp.float32)
    s_step = jnp.sum(rows * m)
    np_step = jnp.sum(m)

    @pl.when(step == 0)
    def _():
        s_ref[...] = jnp.zeros((1, 1), jnp.float32)
        np_ref[...] = jnp.zeros((1, 1), jnp.float32)

    s_ref[...] += s_step.reshape(1, 1)
    np_ref[...] += np_step.reshape(1, 1)


@jax.jit
def kernel(pred, target):
    bsz, n, c = pred.shape
    bc = bsz * c
    bk = bsz * _TOPK

    pred_t64 = jnp.transpose(pred, (0, 2, 1)).reshape(bc, n)
    fp64 = _sc_topk()(pred_t64, target)

    rows = bsz * n
    blk = 4096
    grid = rows // blk
    s, npos = pl.pallas_call(
        _pair_body,
        grid=(grid,),
        in_specs=[
            pl.BlockSpec((blk, c), lambda i: (i, 0)),
            pl.BlockSpec((blk, 1), lambda i: (i, 0)),
            pl.BlockSpec((bc, _TOPK), lambda i: (0, 0)),
        ],
        out_specs=[
            pl.BlockSpec((1, 1), lambda i: (0, 0)),
            pl.BlockSpec((1, 1), lambda i: (0, 0)),
        ],
        out_shape=[
            jax.ShapeDtypeStruct((1, 1), jnp.float32),
            jax.ShapeDtypeStruct((1, 1), jnp.float32),
        ],
    )(pred.reshape(rows, c), target.reshape(rows, 1), fp64)

    denom = npos[0, 0] * float(bk * c)
    return (_LOSS_WEIGHT / _SOFT) * s[0, 0] / denom


# speedup vs baseline: 1.0030x; 1.0030x over previous
import functools

import jax
import jax.numpy as jnp
from jax import lax
from jax.experimental import pallas as pl
from jax.experimental.pallas import tpu as pltpu
from jax.experimental.pallas import tpu_sc as plsc

_DELTA = 0.5
_LOSS_WEIGHT = 0.5
_TOPK = 64
_SOFT = 6.0
_NEG_INF = float("-inf")
_K6 = _SOFT * 1.4426950408889634


def _rev16(v):
    return jnp.flip(v, axis=0)


def _merge16(a, b):
    rb = _rev16(b)
    return jnp.sort(jnp.minimum(a, rb)), jnp.sort(jnp.maximum(a, rb))


def _clean32(x0, x1):
    return jnp.sort(jnp.minimum(x0, x1)), jnp.sort(jnp.maximum(x0, x1))


def _sort64(v0, v1, v2, v3):
    lo0, hi0 = _merge16(jnp.sort(v0), jnp.sort(v1))
    lo1, hi1 = _merge16(jnp.sort(v2), jnp.sort(v3))
    rb0, rb1 = _rev16(hi1), _rev16(lo1)
    loa, lob = jnp.minimum(lo0, rb0), jnp.minimum(hi0, rb1)
    hia, hib = jnp.maximum(lo0, rb0), jnp.maximum(hi0, rb1)
    s0, s1 = _clean32(loa, lob)
    s2, s3 = _clean32(hia, hib)
    return s0, s1, s2, s3


def _merge64_top(a, b):
    z = [
        jnp.maximum(a[0], _rev16(b[3])),
        jnp.maximum(a[1], _rev16(b[2])),
        jnp.maximum(a[2], _rev16(b[1])),
        jnp.maximum(a[3], _rev16(b[0])),
    ]
    t0, t2 = jnp.minimum(z[0], z[2]), jnp.maximum(z[0], z[2])
    t1, t3 = jnp.minimum(z[1], z[3]), jnp.maximum(z[1], z[3])
    u0, u1 = jnp.minimum(t0, t1), jnp.maximum(t0, t1)
    u2, u3 = jnp.minimum(t2, t3), jnp.maximum(t2, t3)
    return jnp.sort(u0), jnp.sort(u1), jnp.sort(u2), jnp.sort(u3)


def _sc_topk_body(pred_hbm, tgt_hbm, out_hbm, prow, trow, bufa, bufb, orow):
    nc_sc = 2
    wid = lax.axis_index("s") * nc_sc + lax.axis_index("c")
    neg_inf = jnp.full((16,), _NEG_INF, jnp.float32)

    def load4(ref, base):
        return [ref[pl.ds(base + 16 * j, 16)] for j in range(4)]

    def store4(ref, base, vs):
        for j in range(4):
            ref[pl.ds(base + 16 * j, 16)] = vs[j]

    for half in range(2):
        r = wid + 32 * half
        b = r // 16
        pltpu.sync_copy(pred_hbm.at[r], prow)
        pltpu.sync_copy(tgt_hbm.at[b], trow)

        def build(i, carry):
            base = i * 64
            vs = []
            for j in range(4):
                v = prow[pl.ds(base + 16 * j, 16)]
                t = trow[pl.ds(base + 16 * j, 16)]
                vs.append(jnp.where(t == 16, v, neg_inf))
            store4(bufa, base, list(_sort64(*vs)))
            return carry

        lax.fori_loop(0, 32, build, 0)

        def make_merge(src, dst):
            def mrg(i, carry):
                a = load4(src, (2 * i) * 64)
                b4 = load4(src, (2 * i + 1) * 64)
                store4(dst, i * 64, list(_merge64_top(a, b4)))
                return carry

            return mrg

        lax.fori_loop(0, 16, make_merge(bufa, bufb), 0)
        lax.fori_loop(0, 8, make_merge(bufb, bufa), 0)
        lax.fori_loop(0, 4, make_merge(bufa, bufb), 0)
        lax.fori_loop(0, 2, make_merge(bufb, bufa), 0)
        lax.fori_loop(0, 1, make_merge(bufa, bufb), 0)

        for j in range(4):
            v = bufb[pl.ds(16 * j, 16)]
            orow[pl.ds(16 * j, 16)] = jnp.where(
                v == _NEG_INF, jnp.nan, _K6 * (v + _DELTA)
            )
        pltpu.sync_copy(orow, out_hbm.at[r])


@functools.cache
def _sc_topk():
    return pl.kernel(
        _sc_topk_body,
        out_type=jax.ShapeDtypeStruct((64, 64), jnp.float32),
        mesh=plsc.VectorSubcoreMesh(core_axis_name="c", subcore_axis_name="s"),
        scratch_types=[
            pltpu.VMEM((2048,), jnp.float32),
            pltpu.VMEM((2048,), jnp.int32),
            pltpu.VMEM((2048,), jnp.float32),
            pltpu.VMEM((1024,), jnp.float32),
            pltpu.VMEM((64,), jnp.float32),
        ],
        compiler_params=pltpu.CompilerParams(needs_layout_passes=False),
    )


def _pair_body(tp_ref, tgt_ref, a2_ref, s_ref, np_ref):
    step = pl.program_id(0)
    r, c = tp_ref.shape
    nb, k = a2_ref.shape
    bk = (nb // c) * k
    tp6 = tp_ref[...] * _K6
    acc = jnp.zeros((r, bk), jnp.float32)
    for cc in range(c):
        a2 = jnp.concatenate(
            [a2_ref[b2 * c + cc : b2 * c + cc + 1, :] for b2 in range(nb // c)],
            axis=1,
        )
        x2 = a2 - tp6[:, cc : cc + 1]
        acc = acc + jnp.log(1.0 + jnp.exp2(x2))
    rows = jnp.sum(acc, axis=1)
    m = (tgt_ref[...][:, 0] != c).astype(jnp.float32)
    s_step = jnp.sum(rows * m)
    np_step = jnp.sum(m)

    @pl.when(step == 0)
    def _():
        s_ref[...] = jnp.zeros((1, 1), jnp.float32)
        np_ref[...] = jnp.zeros((1, 1), jnp.float32)

    s_ref[...] += s_step.reshape(1, 1)
    np_ref[...] += np_step.reshape(1, 1)


@jax.jit
def kernel(pred, target):
    bsz, n, c = pred.shape
    bc = bsz * c
    bk = bsz * _TOPK

    pred_t64 = jnp.transpose(pred, (0, 2, 1)).reshape(bc, n)
    fp64 = _sc_topk()(pred_t64, target)

    rows = bsz * n
    blk = 2048
    grid = rows // blk
    s, npos = pl.pallas_call(
        _pair_body,
        grid=(grid,),
        in_specs=[
            pl.BlockSpec((blk, c), lambda i: (i, 0)),
            pl.BlockSpec((blk, 1), lambda i: (i, 0)),
            pl.BlockSpec((bc, _TOPK), lambda i: (0, 0)),
        ],
        out_specs=[
            pl.BlockSpec((1, 1), lambda i: (0, 0)),
            pl.BlockSpec((1, 1), lambda i: (0, 0)),
        ],
        out_shape=[
            jax.ShapeDtypeStruct((1, 1), jnp.float32),
            jax.ShapeDtypeStruct((1, 1), jnp.float32),
        ],
    )(pred.reshape(rows, c), target.reshape(rows, 1), fp64)

    denom = npos[0, 0] * float(bk * c)
    return (_LOSS_WEIGHT / _SOFT) * s[0, 0] / denom
